# baseline (speedup 1.0000x reference)
import functools

import jax
import jax.numpy as jnp
from jax import lax
from jax.experimental import pallas as pl
from jax.experimental.pallas import tpu as pltpu
from jax.experimental.pallas import tpu_sc as plsc

_N = 10000
_E = 320000
_D = 128

_K = 128
_NTILES = 32
_CH_TOTAL = _E // _K
_CH_PER_TILE = 80
_CH_LAST = _CH_TOTAL - 31 * _CH_PER_TILE
_N_ACC = 10112
_ZR = _N_ACC // 16
_OR_LAST = _N - 15 * _ZR

_mesh = plsc.VectorSubcoreMesh(core_axis_name="c", subcore_axis_name="s")


@functools.partial(
    pl.kernel,
    out_type=jax.ShapeDtypeStruct((2, _N, _D), jnp.float32),
    mesh=_mesh,
    scratch_types=[
        pltpu.VMEM((2, _K), jnp.int32),
        pltpu.VMEM((2, _K), jnp.int32),
        pltpu.VMEM((_K, _D), jnp.float32),
        pltpu.VMEM((_K, _D), jnp.float32),
        pltpu.VMEM_SHARED((_N_ACC, _D), jnp.float32),
        pltpu.SemaphoreType.DMA,
        pltpu.SemaphoreType.DMA,
        pltpu.SemaphoreType.DMA,
        pltpu.SemaphoreType.DMA,
    ],
)
def _sc_aggregate(x_hbm, ei_hbm, zeros_hbm, out_hbm,
                  idx_a, idx_b, rows_a, rows_b, acc_s,
                  sem_ga, sem_gb, sem_ia, sem_ib):
    cid = lax.axis_index("c")
    sid = lax.axis_index("s")
    wid = cid * 16 + sid

    pltpu.sync_copy(zeros_hbm, acc_s.at[pl.ds(sid * _ZR, _ZR)])
    plsc.subcore_barrier()

    t0 = wid * _CH_PER_TILE

    def _start_idx(c, ibuf, isem):
        pltpu.async_copy(ei_hbm.at[:, pl.ds((t0 + c) * _K, _K)], ibuf, isem)

    def _wait_idx(ibuf, isem):
        pltpu.make_async_copy(ei_hbm.at[:, pl.ds(0, _K)], ibuf, isem).wait()

    def _start_gather(ibuf, rbuf, gsem):
        pltpu.async_copy(x_hbm.at[ibuf.at[0]], rbuf, gsem)

    def _wait_gather(rbuf, gsem):
        pltpu.make_async_copy(x_hbm.at[pl.ds(0, _K)], rbuf, gsem).wait()

    def _scatter(ibuf, rbuf):
        pltpu.sync_copy(rbuf, acc_s.at[ibuf.at[1]], add=True)

    def _run_chunks(n):
        _start_idx(0, idx_a, sem_ia)
        _wait_idx(idx_a, sem_ia)
        _start_gather(idx_a, rows_a, sem_ga)
        _start_idx(1, idx_b, sem_ib)

        def body(i, carry):
            c0 = 2 * i
            _wait_idx(idx_b, sem_ib)
            _wait_gather(rows_a, sem_ga)
            _start_gather(idx_b, rows_b, sem_gb)
            _scatter(idx_a, rows_a)

            @pl.when(c0 + 2 < n)
            def _():
                _start_idx(c0 + 2, idx_a, sem_ia)

            _wait_gather(rows_b, sem_gb)

            @pl.when(c0 + 2 < n)
            def _():
                _wait_idx(idx_a, sem_ia)
                _start_gather(idx_a, rows_a, sem_ga)

            _scatter(idx_b, rows_b)

            @pl.when(c0 + 3 < n)
            def _():
                _start_idx(c0 + 3, idx_b, sem_ib)

            return carry

        lax.fori_loop(0, n // 2, body, 0)

    @pl.when(wid < 31)
    def _():
        _run_chunks(_CH_PER_TILE)

    @pl.when(wid == 31)
    def _():
        _run_chunks(_CH_LAST)

    plsc.subcore_barrier()

    @pl.when(sid < 15)
    def _():
        pltpu.sync_copy(acc_s.at[pl.ds(sid * _ZR, _ZR)],
                        out_hbm.at[cid, pl.ds(sid * _ZR, _ZR)])

    @pl.when(sid == 15)
    def _():
        pltpu.sync_copy(acc_s.at[pl.ds(15 * _ZR, _OR_LAST)],
                        out_hbm.at[cid, pl.ds(15 * _ZR, _OR_LAST)])


def _tc_body(p_ref, w_ref, b_ref, o_ref):
    acc = p_ref[0] + p_ref[1]
    h = lax.dot_general(acc, w_ref[...], (((1,), (1,)), ((), ())),
                        preferred_element_type=jnp.float32)
    o_ref[...] = jnp.maximum(h + b_ref[...], 0.0)


_tc_apply = pl.pallas_call(
    _tc_body,
    grid=(10,),
    in_specs=[
        pl.BlockSpec((2, _N // 10, _D), lambda i: (0, i, 0)),
        pl.BlockSpec((_D, _D), lambda i: (0, 0)),
        pl.BlockSpec((1, _D), lambda i: (0, 0)),
    ],
    out_specs=pl.BlockSpec((_N // 10, _D), lambda i: (i, 0)),
    out_shape=jax.ShapeDtypeStruct((_N, _D), jnp.float32),
)


def kernel(x, edge_index, W, b):
    zeros = jnp.zeros((_ZR, _D), jnp.float32)
    partials = _sc_aggregate(x, edge_index, zeros)
    return _tc_apply(partials, W, b.reshape(1, _D))


# speedup vs baseline: 1.1573x; 1.1573x over previous
import functools

import jax
import jax.numpy as jnp
from jax import lax
from jax.experimental import pallas as pl
from jax.experimental.pallas import tpu as pltpu
from jax.experimental.pallas import tpu_sc as plsc

_N = 10000
_E = 320000
_D = 128

_K = 128
_NTILES = 32
_CH_TOTAL = _E // _K
_CH_PER_TILE = 80
_CH_LAST = _CH_TOTAL - 31 * _CH_PER_TILE
_N_ACC = 10112
_ZR = _N_ACC // 16
_OR_LAST = _N - 15 * _ZR

_mesh = plsc.VectorSubcoreMesh(core_axis_name="c", subcore_axis_name="s")


@functools.partial(
    pl.kernel,
    out_type=jax.ShapeDtypeStruct((2, _N, _D), jnp.float32),
    mesh=_mesh,
    scratch_types=[
        pltpu.VMEM((2, _K), jnp.int32),
        pltpu.VMEM((2, _K), jnp.int32),
        pltpu.VMEM((2, _K), jnp.int32),
        pltpu.VMEM((2, _K), jnp.int32),
        pltpu.VMEM((_K, _D), jnp.float32),
        pltpu.VMEM((_K, _D), jnp.float32),
        pltpu.VMEM_SHARED((_N_ACC, _D), jnp.float32),
        pltpu.SemaphoreType.DMA,
        pltpu.SemaphoreType.DMA,
        pltpu.SemaphoreType.DMA,
        pltpu.SemaphoreType.DMA,
        pltpu.SemaphoreType.DMA,
        pltpu.SemaphoreType.DMA,
    ],
)
def _sc_aggregate(x_hbm, ei_hbm, zeros_hbm, out_hbm,
                  idx_0, idx_1, idx_2, idx_3, rows_a, rows_b, acc_s,
                  sem_ga, sem_gb, sem_i0, sem_i1, sem_i2, sem_i3):
    cid = lax.axis_index("c")
    sid = lax.axis_index("s")
    wid = cid * 16 + sid

    pltpu.sync_copy(zeros_hbm, acc_s.at[pl.ds(sid * _ZR, _ZR)])
    plsc.subcore_barrier()

    t0 = wid * _CH_PER_TILE

    def _start_idx(c, ibuf, isem):
        pltpu.async_copy(ei_hbm.at[:, pl.ds((t0 + c) * _K, _K)], ibuf, isem)

    def _wait_idx(ibuf, isem):
        pltpu.make_async_copy(ei_hbm.at[:, pl.ds(0, _K)], ibuf, isem).wait()

    def _start_gather(ibuf, rbuf, gsem):
        pltpu.async_copy(x_hbm.at[ibuf.at[0]], rbuf, gsem)

    def _wait_gather(rbuf, gsem):
        pltpu.make_async_copy(x_hbm.at[pl.ds(0, _K)], rbuf, gsem).wait()

    def _scatter(ibuf, rbuf):
        pltpu.sync_copy(rbuf, acc_s.at[ibuf.at[1]], add=True)

    def _run_chunks(n):
        idx = (idx_0, idx_1, idx_2, idx_3)
        isem = (sem_i0, sem_i1, sem_i2, sem_i3)

        _start_idx(0, idx[0], isem[0])
        _start_idx(1, idx[1], isem[1])
        _wait_idx(idx[0], isem[0])
        _start_gather(idx[0], rows_a, sem_ga)
        _start_idx(2, idx[2], isem[2])
        _start_idx(3, idx[3], isem[3])

        def body(i, carry):
            c0 = 4 * i
            for k in range(4):
                c = c0 + k
                j, j1 = k, (k + 1) % 4
                mine = rows_a if k % 2 == 0 else rows_b
                other = rows_b if k % 2 == 0 else rows_a
                msem = sem_ga if k % 2 == 0 else sem_gb
                osem = sem_gb if k % 2 == 0 else sem_ga

                @pl.when(c + 1 < n)
                def _(j1=j1, other=other, osem=osem):
                    _wait_idx(idx[j1], isem[j1])
                    _start_gather(idx[j1], other, osem)

                _wait_gather(mine, msem)
                _scatter(idx[j], mine)

                @pl.when(c + 4 < n)
                def _(c=c, j=j):
                    _start_idx(c + 4, idx[j], isem[j])

            return carry

        lax.fori_loop(0, n // 4, body, 0)

    @pl.when(wid < 31)
    def _():
        _run_chunks(_CH_PER_TILE)

    @pl.when(wid == 31)
    def _():
        _run_chunks(_CH_LAST)

    plsc.subcore_barrier()

    @pl.when(sid < 15)
    def _():
        pltpu.sync_copy(acc_s.at[pl.ds(sid * _ZR, _ZR)],
                        out_hbm.at[cid, pl.ds(sid * _ZR, _ZR)])

    @pl.when(sid == 15)
    def _():
        pltpu.sync_copy(acc_s.at[pl.ds(15 * _ZR, _OR_LAST)],
                        out_hbm.at[cid, pl.ds(15 * _ZR, _OR_LAST)])


def _tc_body(p_ref, w_ref, b_ref, o_ref):
    acc = p_ref[0] + p_ref[1]
    h = lax.dot_general(acc, w_ref[...], (((1,), (1,)), ((), ())),
                        preferred_element_type=jnp.float32)
    o_ref[...] = jnp.maximum(h + b_ref[...], 0.0)


_tc_apply = pl.pallas_call(
    _tc_body,
    grid=(10,),
    in_specs=[
        pl.BlockSpec((2, _N // 10, _D), lambda i: (0, i, 0)),
        pl.BlockSpec((_D, _D), lambda i: (0, 0)),
        pl.BlockSpec((1, _D), lambda i: (0, 0)),
    ],
    out_specs=pl.BlockSpec((_N // 10, _D), lambda i: (i, 0)),
    out_shape=jax.ShapeDtypeStruct((_N, _D), jnp.float32),
)


def kernel(x, edge_index, W, b):
    zeros = jnp.zeros((_ZR, _D), jnp.float32)
    partials = _sc_aggregate(x, edge_index, zeros)
    return _tc_apply(partials, W, b.reshape(1, _D))


# speedup vs baseline: 1.1972x; 1.0345x over previous
import functools

import jax
import jax.numpy as jnp
from jax import lax
from jax.experimental import pallas as pl
from jax.experimental.pallas import tpu as pltpu
from jax.experimental.pallas import tpu_sc as plsc

_N = 10000
_E = 320000
_D = 128

_K = 128
_NTILES = 32
_CH_TOTAL = _E // _K
_CH_PER_TILE = 80
_CH_LAST = _CH_TOTAL - 31 * _CH_PER_TILE
_N_ACC = 10112
_ZR = _N_ACC // 16
_OR_LAST = _N - 15 * _ZR

_mesh = plsc.VectorSubcoreMesh(core_axis_name="c", subcore_axis_name="s")


@functools.partial(
    pl.kernel,
    out_type=jax.ShapeDtypeStruct((2, _N, _D), jnp.float32),
    mesh=_mesh,
    scratch_types=[
        pltpu.VMEM((2, _K), jnp.int32),
        pltpu.VMEM((2, _K), jnp.int32),
        pltpu.VMEM((2, _K), jnp.int32),
        pltpu.VMEM((2, _K), jnp.int32),
        pltpu.VMEM((_K, _D), jnp.float32),
        pltpu.VMEM((_K, _D), jnp.float32),
        pltpu.VMEM_SHARED((_N_ACC, _D), jnp.float32),
        pltpu.SemaphoreType.DMA,
        pltpu.SemaphoreType.DMA,
        pltpu.SemaphoreType.DMA,
        pltpu.SemaphoreType.DMA,
        pltpu.SemaphoreType.DMA,
        pltpu.SemaphoreType.DMA,
    ],
)
def _sc_aggregate(x_hbm, ei_hbm, out_hbm,
                  idx_0, idx_1, idx_2, idx_3, rows_a, rows_b, acc_s,
                  sem_ga, sem_gb, sem_i0, sem_i1, sem_i2, sem_i3):
    cid = lax.axis_index("c")
    sid = lax.axis_index("s")
    wid = cid * 16 + sid

    zv = jnp.zeros((16,), jnp.float32)

    def _zrow(r, carry):
        for j in range(8):
            rows_a[r, pl.ds(j * 16, 16)] = zv
        return carry

    lax.fori_loop(0, _K, _zrow, 0)
    for i in range(_ZR // _K):
        pltpu.sync_copy(rows_a, acc_s.at[pl.ds(sid * _ZR + i * _K, _K)])
    tail = _ZR % _K
    pltpu.sync_copy(rows_a.at[pl.ds(0, tail)],
                    acc_s.at[pl.ds(sid * _ZR + _ZR - tail, tail)])
    plsc.subcore_barrier()

    t0 = wid * _CH_PER_TILE

    def _start_idx(c, ibuf, isem):
        pltpu.async_copy(ei_hbm.at[:, pl.ds((t0 + c) * _K, _K)], ibuf, isem)

    def _wait_idx(ibuf, isem):
        pltpu.make_async_copy(ei_hbm.at[:, pl.ds(0, _K)], ibuf, isem).wait()

    def _start_gather(ibuf, rbuf, gsem):
        pltpu.async_copy(x_hbm.at[ibuf.at[0]], rbuf, gsem)

    def _wait_gather(rbuf, gsem):
        pltpu.make_async_copy(x_hbm.at[pl.ds(0, _K)], rbuf, gsem).wait()

    def _scatter(ibuf, rbuf):
        pltpu.sync_copy(rbuf, acc_s.at[ibuf.at[1]], add=True)

    def _run_chunks(n):
        idx = (idx_0, idx_1, idx_2, idx_3)
        isem = (sem_i0, sem_i1, sem_i2, sem_i3)

        _start_idx(0, idx[0], isem[0])
        _start_idx(1, idx[1], isem[1])
        _wait_idx(idx[0], isem[0])
        _start_gather(idx[0], rows_a, sem_ga)
        _start_idx(2, idx[2], isem[2])
        _start_idx(3, idx[3], isem[3])

        def body(i, carry):
            c0 = 4 * i
            for k in range(4):
                c = c0 + k
                j, j1 = k, (k + 1) % 4
                mine = rows_a if k % 2 == 0 else rows_b
                other = rows_b if k % 2 == 0 else rows_a
                msem = sem_ga if k % 2 == 0 else sem_gb
                osem = sem_gb if k % 2 == 0 else sem_ga

                @pl.when(c + 1 < n)
                def _(j1=j1, other=other, osem=osem):
                    _wait_idx(idx[j1], isem[j1])
                    _start_gather(idx[j1], other, osem)

                _wait_gather(mine, msem)
                _scatter(idx[j], mine)

                @pl.when(c + 4 < n)
                def _(c=c, j=j):
                    _start_idx(c + 4, idx[j], isem[j])

            return carry

        lax.fori_loop(0, n // 4, body, 0)

    @pl.when(wid < 31)
    def _():
        _run_chunks(_CH_PER_TILE)

    @pl.when(wid == 31)
    def _():
        _run_chunks(_CH_LAST)

    plsc.subcore_barrier()

    @pl.when(sid < 15)
    def _():
        pltpu.sync_copy(acc_s.at[pl.ds(sid * _ZR, _ZR)],
                        out_hbm.at[cid, pl.ds(sid * _ZR, _ZR)])

    @pl.when(sid == 15)
    def _():
        pltpu.sync_copy(acc_s.at[pl.ds(15 * _ZR, _OR_LAST)],
                        out_hbm.at[cid, pl.ds(15 * _ZR, _OR_LAST)])


def _tc_body(p_ref, w_ref, b_ref, o_ref):
    acc = p_ref[0] + p_ref[1]
    h = lax.dot_general(acc, w_ref[...], (((1,), (1,)), ((), ())),
                        preferred_element_type=jnp.float32)
    o_ref[...] = jnp.maximum(h + b_ref[...], 0.0)


_tc_apply = pl.pallas_call(
    _tc_body,
    grid=(10,),
    in_specs=[
        pl.BlockSpec((2, _N // 10, _D), lambda i: (0, i, 0)),
        pl.BlockSpec((_D, _D), lambda i: (0, 0)),
        pl.BlockSpec((1, _D), lambda i: (0, 0)),
    ],
    out_specs=pl.BlockSpec((_N // 10, _D), lambda i: (i, 0)),
    out_shape=jax.ShapeDtypeStruct((_N, _D), jnp.float32),
)


def kernel(x, edge_index, W, b):
    partials = _sc_aggregate(x, edge_index)
    return _tc_apply(partials, W, b.reshape(1, _D))


# speedup vs baseline: 1.2184x; 1.0177x over previous
import functools

import jax
import jax.numpy as jnp
from jax import lax
from jax.experimental import pallas as pl
from jax.experimental.pallas import tpu as pltpu
from jax.experimental.pallas import tpu_sc as plsc

_N = 10000
_E = 320000
_D = 128

_K = 128
_NTILES = 32
_CH_TOTAL = _E // _K
_CH_PER_TILE = 80
_CH_LAST = _CH_TOTAL - 31 * _CH_PER_TILE
_N_ACC = 10112
_ZR = _N_ACC // 16
_OR_LAST = _N - 15 * _ZR

_mesh = plsc.VectorSubcoreMesh(core_axis_name="c", subcore_axis_name="s")


@functools.partial(
    pl.kernel,
    out_type=jax.ShapeDtypeStruct((2, _N, _D), jnp.float32),
    mesh=_mesh,
    scratch_types=[
        pltpu.VMEM((2, _K), jnp.int32),
        pltpu.VMEM((2, _K), jnp.int32),
        pltpu.VMEM((2, _K), jnp.int32),
        pltpu.VMEM((2, _K), jnp.int32),
        pltpu.VMEM((_K, _D), jnp.float32),
        pltpu.VMEM((_K, _D), jnp.float32),
        pltpu.VMEM_SHARED((_N_ACC, _D), jnp.float32),
        pltpu.SemaphoreType.DMA,
        pltpu.SemaphoreType.DMA,
        pltpu.SemaphoreType.DMA,
        pltpu.SemaphoreType.DMA,
        pltpu.SemaphoreType.DMA,
        pltpu.SemaphoreType.DMA,
    ],
)
def _sc_aggregate(x_hbm, ei_hbm, out_hbm,
                  idx_0, idx_1, idx_2, idx_3, rows_a, rows_b, acc_s,
                  sem_ga, sem_gb, sem_i0, sem_i1, sem_i2, sem_i3):
    cid = lax.axis_index("c")
    sid = lax.axis_index("s")
    wid = cid * 16 + sid

    zv = jnp.zeros((16,), jnp.float32)

    def _zrow(r, carry):
        for j in range(8):
            rows_a[r, pl.ds(j * 16, 16)] = zv
        return carry

    lax.fori_loop(0, _K, _zrow, 0)
    for i in range(_ZR // _K):
        pltpu.sync_copy(rows_a, acc_s.at[pl.ds(sid * _ZR + i * _K, _K)])
    tail = _ZR % _K
    pltpu.sync_copy(rows_a.at[pl.ds(0, tail)],
                    acc_s.at[pl.ds(sid * _ZR + _ZR - tail, tail)])
    plsc.subcore_barrier()

    t0 = wid * _CH_PER_TILE

    def _start_idx(c, ibuf, isem):
        pltpu.async_copy(ei_hbm.at[:, pl.ds((t0 + c) * _K, _K)], ibuf, isem)

    def _wait_idx(ibuf, isem):
        pltpu.make_async_copy(ei_hbm.at[:, pl.ds(0, _K)], ibuf, isem).wait()

    def _start_gather(ibuf, rbuf, gsem):
        pltpu.async_copy(x_hbm.at[ibuf.at[0]], rbuf, gsem)

    def _wait_gather(rbuf, gsem):
        pltpu.make_async_copy(x_hbm.at[pl.ds(0, _K)], rbuf, gsem).wait()

    def _scatter(ibuf, rbuf):
        pltpu.sync_copy(rbuf, acc_s.at[ibuf.at[1]], add=True)

    def _run_chunks(n):
        idx = (idx_0, idx_1, idx_2, idx_3)
        isem = (sem_i0, sem_i1, sem_i2, sem_i3)

        _start_idx(0, idx[0], isem[0])
        _start_idx(1, idx[1], isem[1])
        _wait_idx(idx[0], isem[0])
        _start_gather(idx[0], rows_a, sem_ga)
        _start_idx(2, idx[2], isem[2])
        _start_idx(3, idx[3], isem[3])

        def body(i, carry):
            c0 = 4 * i
            for k in range(4):
                c = c0 + k
                j, j1 = k, (k + 1) % 4
                mine = rows_a if k % 2 == 0 else rows_b
                other = rows_b if k % 2 == 0 else rows_a
                msem = sem_ga if k % 2 == 0 else sem_gb
                osem = sem_gb if k % 2 == 0 else sem_ga

                @pl.when(c + 1 < n)
                def _(j1=j1, other=other, osem=osem):
                    _wait_idx(idx[j1], isem[j1])
                    _start_gather(idx[j1], other, osem)

                _wait_gather(mine, msem)
                _scatter(idx[j], mine)

                @pl.when(c + 4 < n)
                def _(c=c, j=j):
                    _start_idx(c + 4, idx[j], isem[j])

            return carry

        lax.fori_loop(0, n // 4, body, 0)

    @pl.when(wid < 31)
    def _():
        _run_chunks(_CH_PER_TILE)

    @pl.when(wid == 31)
    def _():
        _run_chunks(_CH_LAST)

    plsc.subcore_barrier()

    @pl.when(sid < 15)
    def _():
        pltpu.sync_copy(acc_s.at[pl.ds(sid * _ZR, _ZR)],
                        out_hbm.at[cid, pl.ds(sid * _ZR, _ZR)])

    @pl.when(sid == 15)
    def _():
        pltpu.sync_copy(acc_s.at[pl.ds(15 * _ZR, _OR_LAST)],
                        out_hbm.at[cid, pl.ds(15 * _ZR, _OR_LAST)])


def _tc_body(p_ref, w_ref, b_ref, o_ref):
    acc = p_ref[0] + p_ref[1]
    h = lax.dot_general(acc, w_ref[...], (((1,), (1,)), ((), ())),
                        preferred_element_type=jnp.float32)
    o_ref[...] = jnp.maximum(h + b_ref[...], 0.0)


_TC_G = 5
_tc_apply = pl.pallas_call(
    _tc_body,
    grid=(_TC_G,),
    in_specs=[
        pl.BlockSpec((2, _N // _TC_G, _D), lambda i: (0, i, 0)),
        pl.BlockSpec((_D, _D), lambda i: (0, 0)),
        pl.BlockSpec((1, _D), lambda i: (0, 0)),
    ],
    out_specs=pl.BlockSpec((_N // _TC_G, _D), lambda i: (i, 0)),
    out_shape=jax.ShapeDtypeStruct((_N, _D), jnp.float32),
)


def kernel(x, edge_index, W, b):
    partials = _sc_aggregate(x, edge_index)
    return _tc_apply(partials, W, b.reshape(1, _D))


# speedup vs baseline: 1.2470x; 1.0235x over previous
import functools

import jax
import jax.numpy as jnp
from jax import lax
from jax.experimental import pallas as pl
from jax.experimental.pallas import tpu as pltpu
from jax.experimental.pallas import tpu_sc as plsc

_N = 10000
_E = 320000
_D = 128

_K = 128
_NTILES = 32
_CH_TOTAL = _E // _K
_CH_PER_TILE = 80
_CH_LAST = _CH_TOTAL - 31 * _CH_PER_TILE
_N_ACC = 10112
_ZR = _N_ACC // 16
_OR_LAST = _N - 15 * _ZR

_mesh = plsc.VectorSubcoreMesh(core_axis_name="c", subcore_axis_name="s")


@functools.partial(
    pl.kernel,
    out_type=jax.ShapeDtypeStruct((2, _N, _D), jnp.float32),
    mesh=_mesh,
    scratch_types=[
        pltpu.VMEM((2, _K), jnp.int32),
        pltpu.VMEM((2, _K), jnp.int32),
        pltpu.VMEM((2, _K), jnp.int32),
        pltpu.VMEM((2, _K), jnp.int32),
        pltpu.VMEM((_K, _D), jnp.float32),
        pltpu.VMEM((_K, _D), jnp.float32),
        pltpu.VMEM_SHARED((_N_ACC, _D), jnp.float32),
        pltpu.SemaphoreType.DMA,
        pltpu.SemaphoreType.DMA,
        pltpu.SemaphoreType.DMA,
        pltpu.SemaphoreType.DMA,
        pltpu.SemaphoreType.DMA,
        pltpu.SemaphoreType.DMA,
    ],
)
def _sc_aggregate(x_hbm, ei_hbm, out_hbm,
                  idx_0, idx_1, idx_2, idx_3, rows_a, rows_b, acc_s,
                  sem_ga, sem_gb, sem_i0, sem_i1, sem_i2, sem_i3):
    cid = lax.axis_index("c")
    sid = lax.axis_index("s")
    wid = cid * 16 + sid

    zv = jnp.zeros((16,), jnp.float32)

    def _zrow(r, carry):
        for j in range(8):
            rows_a[r, pl.ds(j * 16, 16)] = zv
        return carry

    lax.fori_loop(0, _K, _zrow, 0)
    for i in range(_ZR // _K):
        pltpu.sync_copy(rows_a, acc_s.at[pl.ds(sid * _ZR + i * _K, _K)])
    tail = _ZR % _K
    pltpu.sync_copy(rows_a.at[pl.ds(0, tail)],
                    acc_s.at[pl.ds(sid * _ZR + _ZR - tail, tail)])
    plsc.subcore_barrier()

    t0 = wid * _CH_PER_TILE

    def _start_idx(c, ibuf, isem):
        pltpu.async_copy(ei_hbm.at[:, pl.ds((t0 + c) * _K, _K)], ibuf, isem)

    def _wait_idx(ibuf, isem):
        pltpu.make_async_copy(ei_hbm.at[:, pl.ds(0, _K)], ibuf, isem).wait()

    def _start_gather(ibuf, rbuf, gsem):
        pltpu.async_copy(x_hbm.at[ibuf.at[0]], rbuf, gsem)

    def _wait_gather(rbuf, gsem):
        pltpu.make_async_copy(x_hbm.at[pl.ds(0, _K)], rbuf, gsem).wait()

    def _scatter(ibuf, rbuf):
        pltpu.sync_copy(rbuf, acc_s.at[ibuf.at[1]], add=True)

    def _run_chunks(n):
        idx = (idx_0, idx_1, idx_2, idx_3)
        isem = (sem_i0, sem_i1, sem_i2, sem_i3)

        _start_idx(0, idx[0], isem[0])
        _start_idx(1, idx[1], isem[1])
        _wait_idx(idx[0], isem[0])
        _start_gather(idx[0], rows_a, sem_ga)
        _start_idx(2, idx[2], isem[2])
        _start_idx(3, idx[3], isem[3])

        def body(i, carry):
            c0 = 4 * i
            for k in range(4):
                c = c0 + k
                j, j1 = k, (k + 1) % 4
                mine = rows_a if k % 2 == 0 else rows_b
                other = rows_b if k % 2 == 0 else rows_a
                msem = sem_ga if k % 2 == 0 else sem_gb
                osem = sem_gb if k % 2 == 0 else sem_ga

                @pl.when(c + 1 < n)
                def _(j1=j1, other=other, osem=osem):
                    _wait_idx(idx[j1], isem[j1])
                    _start_gather(idx[j1], other, osem)

                _wait_gather(mine, msem)
                _scatter(idx[j], mine)

                @pl.when(c + 4 < n)
                def _(c=c, j=j):
                    _start_idx(c + 4, idx[j], isem[j])

            return carry

        lax.fori_loop(0, n // 4, body, 0)

    @pl.when(wid < 31)
    def _():
        _run_chunks(_CH_PER_TILE)

    @pl.when(wid == 31)
    def _():
        _run_chunks(_CH_LAST)

    plsc.subcore_barrier()

    @pl.when(sid < 15)
    def _():
        pltpu.sync_copy(acc_s.at[pl.ds(sid * _ZR, _ZR)],
                        out_hbm.at[cid, pl.ds(sid * _ZR, _ZR)])

    @pl.when(sid == 15)
    def _():
        pltpu.sync_copy(acc_s.at[pl.ds(15 * _ZR, _OR_LAST)],
                        out_hbm.at[cid, pl.ds(15 * _ZR, _OR_LAST)])


def _tc_body(p_ref, w_ref, b_ref, o_ref):
    acc = p_ref[0] + p_ref[1]
    h = lax.dot_general(acc, w_ref[...], (((1,), (1,)), ((), ())),
                        preferred_element_type=jnp.float32)
    o_ref[...] = jnp.maximum(h + b_ref[...], 0.0)


_TC_G = 2
_tc_apply = pl.pallas_call(
    _tc_body,
    grid=(_TC_G,),
    in_specs=[
        pl.BlockSpec((2, _N // _TC_G, _D), lambda i: (0, i, 0)),
        pl.BlockSpec((_D, _D), lambda i: (0, 0)),
        pl.BlockSpec((1, _D), lambda i: (0, 0)),
    ],
    out_specs=pl.BlockSpec((_N // _TC_G, _D), lambda i: (i, 0)),
    out_shape=jax.ShapeDtypeStruct((_N, _D), jnp.float32),
)


def kernel(x, edge_index, W, b):
    partials = _sc_aggregate(x, edge_index)
    return _tc_apply(partials, W, b.reshape(1, _D))
